# baseline (speedup 1.0000x reference)
import functools

import jax
import jax.numpy as jnp
from jax import lax
from jax.experimental import pallas as pl
from jax.experimental.pallas import tpu as pltpu
from jax.experimental.pallas import tpu_sc as plsc

NUM_CLASSES = 100000
DIM = 128
BATCH = 16384
NC = 2
NS = 16
NW = NC * NS
ROWS_PER_W = BATCH // NW
SUB = 128
NSUB = ROWS_PER_W // SUB
NBUF = 2
HIST_PER_TILE = 6272
HIST_PAD = NS * HIST_PER_TILE
Y_PER_TILE = BATCH // NS

_mesh = plsc.VectorSubcoreMesh(core_axis_name="c", subcore_axis_name="s")


@functools.partial(
    pl.kernel,
    out_type=jax.ShapeDtypeStruct((NW, 16), jnp.float32),
    mesh=_mesh,
    scratch_types=[
        pltpu.VMEM((HIST_PER_TILE,), jnp.float32),
        pltpu.VMEM((Y_PER_TILE,), jnp.float32),
        pltpu.VMEM((Y_PER_TILE,), jnp.int32),
        pltpu.VMEM((ROWS_PER_W,), jnp.int32),
        pltpu.VMEM((ROWS_PER_W,), jnp.float32),
        pltpu.VMEM((ROWS_PER_W,), jnp.float32),
        pltpu.VMEM((NBUF, SUB, DIM), jnp.float32),
        pltpu.VMEM((NBUF, SUB * DIM), jnp.float32),
        pltpu.VMEM((16,), jnp.float32),
        pltpu.VMEM_SHARED((HIST_PAD,), jnp.float32),
        pltpu.SemaphoreType.DMA,
        pltpu.SemaphoreType.DMA,
    ],
)
def _center_loss_sc(y_hbm, hidden_hbm, centers_hbm, out_hbm,
                    zbuf, ones_v, ych, idx_v, cnt_v, inv_v, crows, hrows,
                    tv, hist, sem_c, sem_h):
    cid = lax.axis_index("c")
    sid = lax.axis_index("s")
    wid = cid * NS + sid
    base = wid * ROWS_PER_W

    zeros16 = jnp.zeros((16,), jnp.float32)
    ones16 = jnp.ones((16,), jnp.float32)

    pltpu.sync_copy(y_hbm.at[pl.ds(base, ROWS_PER_W)], idx_v)

    def start_chunk(t):
        buf = t % NBUF
        pltpu.async_copy(
            centers_hbm.at[idx_v.at[pl.ds(t * SUB, SUB)]], crows.at[buf], sem_c)
        pltpu.async_copy(
            hidden_hbm.at[pl.ds((base + t * SUB) * DIM, SUB * DIM)],
            hrows.at[buf], sem_h)

    def wait_chunk(t):
        buf = t % NBUF
        pltpu.make_async_copy(
            centers_hbm.at[idx_v.at[pl.ds(t * SUB, SUB)]], crows.at[buf],
            sem_c).wait()
        pltpu.make_async_copy(
            hidden_hbm.at[pl.ds((base + t * SUB) * DIM, SUB * DIM)],
            hrows.at[buf], sem_h).wait()

    for t in range(NBUF):
        start_chunk(t)

    with jax.named_scope("fills"):
        def fill_z(i, carry):
            zbuf[pl.ds(i * 16, 16)] = zeros16
            return carry

        lax.fori_loop(0, HIST_PER_TILE // 16, fill_z, 0, unroll=8)

        def fill_o(i, carry):
            ones_v[pl.ds(i * 16, 16)] = ones16
            return carry

        lax.fori_loop(0, Y_PER_TILE // 16, fill_o, 0, unroll=8)

    with jax.named_scope("hist"):
        pltpu.sync_copy(zbuf, hist.at[pl.ds(sid * HIST_PER_TILE, HIST_PER_TILE)])
        pltpu.sync_copy(y_hbm.at[pl.ds(sid * Y_PER_TILE, Y_PER_TILE)], ych)
        plsc.subcore_barrier()
        pltpu.sync_copy(ones_v, hist.at[ych], add=True)
        plsc.subcore_barrier()

    with jax.named_scope("counts"):
        pltpu.sync_copy(hist.at[idx_v], cnt_v)

        def fill_inv(i, carry):
            c16 = cnt_v[pl.ds(i * 16, 16)]
            inv_v[pl.ds(i * 16, 16)] = 0.5 / (c16 + 1.0)
            return carry

        lax.fori_loop(0, ROWS_PER_W // 16, fill_inv, 0, unroll=8)

    total = (zeros16, zeros16, zeros16, zeros16)
    for t in range(NSUB):
        with jax.named_scope(f"wait{t}"):
            wait_chunk(t)
        if t + NBUF < NSUB:
            start_chunk(t + NBUF)
        buf = t % NBUF
        cbuf = crows.at[buf]
        hbuf = hrows.at[buf]
        inv_base = t * SUB

        with jax.named_scope(f"compute{t}"):
            @plsc.parallel_loop(0, SUB // 16, carry=total)
            def group_body(g, tots):
                goff = pl.multiple_of(g * (16 * DIM), DIM)
                inv16 = inv_v[pl.ds(inv_base + g * 16, 16)]
                t0, t1, t2, t3 = tots
                rtots = [t0, t1, t2, t3]
                for rr in range(16):
                    roff = rr * DIM
                    row = g * 16 + rr
                    sq = []
                    for k in range(DIM // 16):
                        hv = hbuf[pl.ds(goff + roff + k * 16, 16)]
                        cv = cbuf[row, pl.ds(k * 16, 16)]
                        d = hv - cv
                        sq.append(d * d)
                    acc = ((sq[0] + sq[1]) + (sq[2] + sq[3])) + \
                          ((sq[4] + sq[5]) + (sq[6] + sq[7]))
                    inv_r = lax.gather(
                        inv16, jnp.full((16, 1), rr, jnp.int32),
                        lax.GatherDimensionNumbers(
                            offset_dims=(), collapsed_slice_dims=(0,),
                            start_index_map=(0,)),
                        slice_sizes=(1,),
                        mode=lax.GatherScatterMode.PROMISE_IN_BOUNDS)
                    rtots[rr % 4] = rtots[rr % 4] + acc * inv_r
                return tuple(rtots)

            total = group_body

    tv[...] = (total[0] + total[1]) + (total[2] + total[3])
    pltpu.sync_copy(tv, out_hbm.at[wid])


def kernel(y, hidden, centers):
    parts = _center_loss_sc(y.astype(jnp.int32), hidden.reshape(-1), centers)
    return jnp.sum(parts)


# speedup vs baseline: 1.0171x; 1.0171x over previous
import functools

import jax
import jax.numpy as jnp
from jax import lax
from jax.experimental import pallas as pl
from jax.experimental.pallas import tpu as pltpu
from jax.experimental.pallas import tpu_sc as plsc

NUM_CLASSES = 100000
DIM = 128
BATCH = 16384
NC = 2
NS = 16
NW = NC * NS
ROWS_PER_W = BATCH // NW
SUB = 128
NSUB = ROWS_PER_W // SUB
NBUF = 2
HIST_PER_TILE = 6272
HIST_PAD = NS * HIST_PER_TILE
Y_PER_TILE = BATCH // NS

_mesh = plsc.VectorSubcoreMesh(core_axis_name="c", subcore_axis_name="s")


@functools.partial(
    pl.kernel,
    out_type=jax.ShapeDtypeStruct((NW, 16), jnp.float32),
    mesh=_mesh,
    scratch_types=[
        pltpu.VMEM((HIST_PER_TILE,), jnp.float32),
        pltpu.VMEM((Y_PER_TILE,), jnp.float32),
        pltpu.VMEM((Y_PER_TILE,), jnp.int32),
        pltpu.VMEM((ROWS_PER_W,), jnp.int32),
        pltpu.VMEM((ROWS_PER_W,), jnp.float32),
        pltpu.VMEM((ROWS_PER_W,), jnp.float32),
        pltpu.VMEM((NBUF, SUB, DIM), jnp.float32),
        pltpu.VMEM((NBUF, SUB, DIM), jnp.float32),
        pltpu.VMEM((16,), jnp.float32),
        pltpu.VMEM_SHARED((HIST_PAD,), jnp.float32),
        pltpu.SemaphoreType.DMA,
        pltpu.SemaphoreType.DMA,
    ],
)
def _center_loss_sc(y_hbm, hidden_hbm, centers_hbm, out_hbm,
                    zbuf, ones_v, ych, idx_v, cnt_v, inv_v, crows, hrows,
                    tv, hist, sem_c, sem_h):
    cid = lax.axis_index("c")
    sid = lax.axis_index("s")
    wid = cid * NS + sid
    base = wid * ROWS_PER_W

    zeros16 = jnp.zeros((16,), jnp.float32)
    ones16 = jnp.ones((16,), jnp.float32)

    pltpu.sync_copy(y_hbm.at[pl.ds(base, ROWS_PER_W)], idx_v)

    def start_chunk(t):
        buf = t % NBUF
        pltpu.async_copy(
            centers_hbm.at[idx_v.at[pl.ds(t * SUB, SUB)]], crows.at[buf], sem_c)
        pltpu.async_copy(
            hidden_hbm.at[pl.ds(base + t * SUB, SUB)], hrows.at[buf], sem_h)

    def wait_chunk(t):
        buf = t % NBUF
        pltpu.make_async_copy(
            centers_hbm.at[idx_v.at[pl.ds(t * SUB, SUB)]], crows.at[buf],
            sem_c).wait()
        pltpu.make_async_copy(
            hidden_hbm.at[pl.ds(base + t * SUB, SUB)], hrows.at[buf],
            sem_h).wait()

    for t in range(NBUF):
        start_chunk(t)

    with jax.named_scope("fills"):
        def fill_z(i, carry):
            zbuf[pl.ds(i * 16, 16)] = zeros16
            return carry

        lax.fori_loop(0, HIST_PER_TILE // 16, fill_z, 0, unroll=8)

        def fill_o(i, carry):
            ones_v[pl.ds(i * 16, 16)] = ones16
            return carry

        lax.fori_loop(0, Y_PER_TILE // 16, fill_o, 0, unroll=8)

    with jax.named_scope("hist"):
        pltpu.sync_copy(zbuf, hist.at[pl.ds(sid * HIST_PER_TILE, HIST_PER_TILE)])
        pltpu.sync_copy(y_hbm.at[pl.ds(sid * Y_PER_TILE, Y_PER_TILE)], ych)
        plsc.subcore_barrier()
        pltpu.sync_copy(ones_v, hist.at[ych], add=True)
        plsc.subcore_barrier()

    with jax.named_scope("counts"):
        pltpu.sync_copy(hist.at[idx_v], cnt_v)

        def fill_inv(i, carry):
            c16 = cnt_v[pl.ds(i * 16, 16)]
            inv_v[pl.ds(i * 16, 16)] = 0.5 / (c16 + 1.0)
            return carry

        lax.fori_loop(0, ROWS_PER_W // 16, fill_inv, 0, unroll=8)

    total = zeros16
    for t in range(NSUB):
        with jax.named_scope(f"wait{t}"):
            wait_chunk(t)
        if t + NBUF < NSUB:
            start_chunk(t + NBUF)
        buf = t % NBUF
        cbuf = crows.at[buf]
        hbuf = hrows.at[buf]
        inv_base = t * SUB

        def group_body(g, carry):
            tot0, tot1 = carry
            inv16 = inv_v[pl.ds(inv_base + g * 16, 16)]
            for rr in range(0, 16, 2):
                ra = g * 16 + rr
                rb = ra + 1
                acc_a = zeros16
                acc_b = zeros16
                for k in range(DIM // 16):
                    ha = hbuf[ra, pl.ds(k * 16, 16)]
                    ca = cbuf[ra, pl.ds(k * 16, 16)]
                    hb = hbuf[rb, pl.ds(k * 16, 16)]
                    cb = cbuf[rb, pl.ds(k * 16, 16)]
                    da = ha - ca
                    db = hb - cb
                    acc_a = acc_a + da * da
                    acc_b = acc_b + db * db
                dn = lax.GatherDimensionNumbers(
                    offset_dims=(), collapsed_slice_dims=(0,),
                    start_index_map=(0,))
                inv_a = lax.gather(
                    inv16, jnp.full((16, 1), rr, jnp.int32), dn,
                    slice_sizes=(1,),
                    mode=lax.GatherScatterMode.PROMISE_IN_BOUNDS)
                inv_b = lax.gather(
                    inv16, jnp.full((16, 1), rr + 1, jnp.int32), dn,
                    slice_sizes=(1,),
                    mode=lax.GatherScatterMode.PROMISE_IN_BOUNDS)
                tot0 = tot0 + acc_a * inv_a
                tot1 = tot1 + acc_b * inv_b
            return (tot0, tot1)

        with jax.named_scope(f"compute{t}"):
            total2 = lax.fori_loop(0, SUB // 16, group_body, (total, zeros16))
            total = total2[0] + total2[1]

    tv[...] = total
    pltpu.sync_copy(tv, out_hbm.at[wid])


def kernel(y, hidden, centers):
    parts = _center_loss_sc(y.astype(jnp.int32), hidden, centers)
    return jnp.sum(parts)


# speedup vs baseline: 1.1962x; 1.1762x over previous
import functools

import jax
import jax.numpy as jnp
from jax import lax
from jax.experimental import pallas as pl
from jax.experimental.pallas import tpu as pltpu
from jax.experimental.pallas import tpu_sc as plsc

NUM_CLASSES = 100000
DIM = 128
BATCH = 16384
NC = 2
NS = 16
NW = NC * NS
ROWS_PER_W = BATCH // NW
SUB = 128
NSUB = ROWS_PER_W // SUB
HIST_PER_TILE = 6272
HIST_PAD = NS * HIST_PER_TILE
Y_PER_TILE = BATCH // NS

_mesh = plsc.VectorSubcoreMesh(core_axis_name="c", subcore_axis_name="s")


@functools.partial(
    pl.kernel,
    out_type=(jax.ShapeDtypeStruct((BATCH, DIM), jnp.float32),
              jax.ShapeDtypeStruct((NW, ROWS_PER_W), jnp.float32)),
    mesh=_mesh,
    scratch_types=[
        pltpu.VMEM((HIST_PER_TILE,), jnp.float32),
        pltpu.VMEM((Y_PER_TILE,), jnp.float32),
        pltpu.VMEM((Y_PER_TILE,), jnp.int32),
        pltpu.VMEM((ROWS_PER_W,), jnp.int32),
        pltpu.VMEM((ROWS_PER_W,), jnp.float32),
        pltpu.VMEM((ROWS_PER_W,), jnp.float32),
        pltpu.VMEM((ROWS_PER_W, DIM), jnp.float32),
        pltpu.VMEM_SHARED((HIST_PAD,), jnp.float32),
        pltpu.SemaphoreType.DMA,
    ],
)
def _sc_stage(y_hbm, centers_hbm, gat_hbm, inv_hbm,
              zbuf, ones_v, ych, idx_v, cnt_v, inv_v, gbuf, hist, sem_g):
    cid = lax.axis_index("c")
    sid = lax.axis_index("s")
    wid = cid * NS + sid
    base = wid * ROWS_PER_W

    zeros16 = jnp.zeros((16,), jnp.float32)
    ones16 = jnp.ones((16,), jnp.float32)

    pltpu.sync_copy(y_hbm.at[pl.ds(base, ROWS_PER_W)], idx_v)
    for t in range(NSUB):
        pltpu.async_copy(
            centers_hbm.at[idx_v.at[pl.ds(t * SUB, SUB)]],
            gbuf.at[pl.ds(t * SUB, SUB)], sem_g)

    with jax.named_scope("fills"):
        def fill_z(i, carry):
            zbuf[pl.ds(i * 16, 16)] = zeros16
            return carry

        lax.fori_loop(0, HIST_PER_TILE // 16, fill_z, 0, unroll=8)

        def fill_o(i, carry):
            ones_v[pl.ds(i * 16, 16)] = ones16
            return carry

        lax.fori_loop(0, Y_PER_TILE // 16, fill_o, 0, unroll=8)

    with jax.named_scope("hist"):
        pltpu.sync_copy(zbuf, hist.at[pl.ds(sid * HIST_PER_TILE, HIST_PER_TILE)])
        pltpu.sync_copy(y_hbm.at[pl.ds(sid * Y_PER_TILE, Y_PER_TILE)], ych)
        plsc.subcore_barrier()
        pltpu.sync_copy(ones_v, hist.at[ych], add=True)
        plsc.subcore_barrier()

    with jax.named_scope("counts"):
        pltpu.sync_copy(hist.at[idx_v], cnt_v)

        def fill_inv(i, carry):
            c16 = cnt_v[pl.ds(i * 16, 16)]
            inv_v[pl.ds(i * 16, 16)] = 0.5 / (c16 + 1.0)
            return carry

        lax.fori_loop(0, ROWS_PER_W // 16, fill_inv, 0, unroll=8)
        pltpu.sync_copy(inv_v, inv_hbm.at[wid])

    with jax.named_scope("drain"):
        for t in range(NSUB):
            pltpu.make_async_copy(
                centers_hbm.at[idx_v.at[pl.ds(t * SUB, SUB)]],
                gbuf.at[pl.ds(t * SUB, SUB)], sem_g).wait()
        pltpu.sync_copy(gbuf, gat_hbm.at[pl.ds(base, ROWS_PER_W)])


def _tc_body(h_ref, g_ref, iv_ref, o_ref):
    d = h_ref[...] - g_ref[...]
    s = jnp.sum(d * d, axis=2)
    p = jnp.sum(s * iv_ref[0])

    @pl.when(pl.program_id(0) == 0)
    def _():
        o_ref[0, 0] = 0.0

    o_ref[0, 0] += p


_GRID = 32
_RB = BATCH // DIM // _GRID

_tc_loss = pl.pallas_call(
    _tc_body,
    grid=(_GRID,),
    in_specs=[
        pl.BlockSpec((_RB, DIM, DIM), lambda i: (i, 0, 0)),
        pl.BlockSpec((_RB, DIM, DIM), lambda i: (i, 0, 0)),
        pl.BlockSpec((1, _RB, DIM), lambda i: (i, 0, 0)),
    ],
    out_specs=pl.BlockSpec(memory_space=pltpu.SMEM),
    out_shape=jax.ShapeDtypeStruct((1, 1), jnp.float32),
)


def kernel(y, hidden, centers):
    gat, inv = _sc_stage(y.astype(jnp.int32), centers)
    h3 = hidden.reshape(BATCH // DIM, DIM, DIM)
    g3 = gat.reshape(BATCH // DIM, DIM, DIM)
    iv2 = inv.reshape(_GRID, _RB, DIM)
    out = _tc_loss(h3, g3, iv2)
    return out[0, 0]


# speedup vs baseline: 1.3147x; 1.0991x over previous
import functools

import jax
import jax.numpy as jnp
from jax import lax
from jax.experimental import pallas as pl
from jax.experimental.pallas import tpu as pltpu
from jax.experimental.pallas import tpu_sc as plsc

NUM_CLASSES = 100000
DIM = 128
BATCH = 16384
NC = 2
NS = 16
NW = NC * NS
ROWS_PER_W = BATCH // NW
SUB = 128
NSUB = ROWS_PER_W // SUB
HIST_PER_TILE = 6272
HIST_PAD = NS * HIST_PER_TILE
Y_PER_TILE = BATCH // NS

_mesh = plsc.VectorSubcoreMesh(core_axis_name="c", subcore_axis_name="s")


@functools.partial(
    pl.kernel,
    out_type=(jax.ShapeDtypeStruct((BATCH, DIM), jnp.float32),
              jax.ShapeDtypeStruct((NW, ROWS_PER_W), jnp.float32)),
    mesh=_mesh,
    scratch_types=[
        pltpu.VMEM((HIST_PER_TILE,), jnp.float32),
        pltpu.VMEM((Y_PER_TILE,), jnp.float32),
        pltpu.VMEM((Y_PER_TILE,), jnp.int32),
        pltpu.VMEM((ROWS_PER_W,), jnp.int32),
        pltpu.VMEM((ROWS_PER_W,), jnp.float32),
        pltpu.VMEM((ROWS_PER_W,), jnp.float32),
        pltpu.VMEM((ROWS_PER_W, DIM), jnp.float32),
        pltpu.VMEM_SHARED((HIST_PAD,), jnp.float32),
        pltpu.SemaphoreType.DMA,
    ],
)
def _sc_stage(y_hbm, centers_hbm, gat_hbm, inv_hbm,
              zbuf, ones_v, ych, idx_v, cnt_v, inv_v, gbuf, hist, sem_g):
    cid = lax.axis_index("c")
    sid = lax.axis_index("s")
    wid = cid * NS + sid
    base = wid * ROWS_PER_W

    zeros16 = jnp.zeros((16,), jnp.float32)
    ones16 = jnp.ones((16,), jnp.float32)

    pltpu.sync_copy(y_hbm.at[pl.ds(base, ROWS_PER_W)], idx_v)
    for t in range(NSUB):
        pltpu.async_copy(
            centers_hbm.at[idx_v.at[pl.ds(t * SUB, SUB)]],
            gbuf.at[pl.ds(t * SUB, SUB)], sem_g)

    with jax.named_scope("fills"):
        def fill_z(i, carry):
            zbuf[pl.ds(i * 16, 16)] = zeros16
            return carry

        lax.fori_loop(0, HIST_PER_TILE // 16, fill_z, 0, unroll=8)

        def fill_o(i, carry):
            ones_v[pl.ds(i * 16, 16)] = ones16
            return carry

        lax.fori_loop(0, Y_PER_TILE // 16, fill_o, 0, unroll=8)

    with jax.named_scope("hist"):
        pltpu.sync_copy(zbuf, hist.at[pl.ds(sid * HIST_PER_TILE, HIST_PER_TILE)])
        pltpu.sync_copy(y_hbm.at[pl.ds(sid * Y_PER_TILE, Y_PER_TILE)], ych)
        plsc.subcore_barrier()
        pltpu.sync_copy(ones_v, hist.at[ych], add=True)
        plsc.subcore_barrier()

    with jax.named_scope("counts"):
        pltpu.sync_copy(hist.at[idx_v], cnt_v)

        def fill_inv(i, carry):
            c16 = cnt_v[pl.ds(i * 16, 16)]
            inv_v[pl.ds(i * 16, 16)] = 0.5 / (c16 + 1.0)
            return carry

        lax.fori_loop(0, ROWS_PER_W // 16, fill_inv, 0, unroll=8)
        pltpu.sync_copy(inv_v, inv_hbm.at[wid])

    with jax.named_scope("drain"):
        for t in range(NSUB):
            pltpu.make_async_copy(
                centers_hbm.at[idx_v.at[pl.ds(t * SUB, SUB)]],
                gbuf.at[pl.ds(t * SUB, SUB)], sem_g).wait()
        pltpu.sync_copy(gbuf, gat_hbm.at[pl.ds(base, ROWS_PER_W)])


def _tc_body(h_ref, g_ref, iv_ref, acc_ref, o_ref):
    d = h_ref[...] - g_ref[...]
    w = d * d * iv_ref[0][:, :, None]
    p = w[0] + w[1] + w[2] + w[3]

    @pl.when(pl.program_id(0) == 0)
    def _():
        acc_ref[...] = jnp.zeros_like(acc_ref)

    acc_ref[...] += p

    @pl.when(pl.program_id(0) == _GRID - 1)
    def _():
        o_ref[0, 0] = jnp.sum(acc_ref[...])


_GRID = 32
_RB = BATCH // DIM // _GRID

_tc_loss = pl.pallas_call(
    _tc_body,
    grid=(_GRID,),
    in_specs=[
        pl.BlockSpec((_RB, DIM, DIM), lambda i: (i, 0, 0)),
        pl.BlockSpec((_RB, DIM, DIM), lambda i: (i, 0, 0)),
        pl.BlockSpec((1, _RB, DIM), lambda i: (i, 0, 0)),
    ],
    out_specs=[pl.BlockSpec((DIM, DIM), lambda i: (0, 0)),
               pl.BlockSpec(memory_space=pltpu.SMEM)],
    out_shape=[jax.ShapeDtypeStruct((DIM, DIM), jnp.float32),
               jax.ShapeDtypeStruct((1, 1), jnp.float32)],
)


def kernel(y, hidden, centers):
    gat, inv = _sc_stage(y.astype(jnp.int32), centers)
    h3 = hidden.reshape(BATCH // DIM, DIM, DIM)
    g3 = gat.reshape(BATCH // DIM, DIM, DIM)
    iv2 = inv.reshape(_GRID, _RB, DIM)
    _, out = _tc_loss(h3, g3, iv2)
    return out[0, 0]
